# baseline (speedup 1.0000x reference)
import functools

import jax
import jax.numpy as jnp
from jax import lax
from jax.experimental import pallas as pl
from jax.experimental.pallas import tpu as pltpu
from jax.experimental.pallas import tpu_sc as plsc

_N = 10000
_NPAD = 10112
_E1 = 330000
_EPAD = 330240
_F = 128
_HID = 128
_HEADS = 8
_NCLS = 3

_RB = 400
_RBP = 632
_RBT = 2528

_NCHUNK = 12
_CROWS = 896
_ACC_ROWS = _CROWS * _HEADS
_YROWS = _NCHUNK * _ACC_ROWS

_EB = 240
_EB2 = 120
_G = 48

_mesh = functools.partial(
    plsc.VectorSubcoreMesh, core_axis_name="c", subcore_axis_name="s"
)


def _l1_alpha_kernel(x_ref, w_ref, asrc_ref, adst_ref, as_ref, ad_ref):
    h = jnp.dot(x_ref[...], w_ref[...], preferred_element_type=jnp.float32)
    r = h.shape[0]
    h3 = h.reshape(r, _HEADS, _HID)
    a = jnp.sum(h3 * asrc_ref[...][None, :, :], axis=-1)
    d = jnp.sum(h3 * adst_ref[...][None, :, :], axis=-1)
    z = jnp.zeros((r, 112), jnp.float32)
    as_ref[...] = jnp.concatenate([a, a, z], axis=1)
    ad_ref[...] = jnp.concatenate([d, d, z], axis=1)


def _l1_alpha(x, W1, att_src1, att_dst1):
    return pl.pallas_call(
        _l1_alpha_kernel,
        grid=(_N // _RB,),
        in_specs=[
            pl.BlockSpec((_RB, _F), lambda i: (i, 0)),
            pl.BlockSpec((_F, _HEADS * _HID), lambda i: (0, 0)),
            pl.BlockSpec((_HEADS, _HID), lambda i: (0, 0)),
            pl.BlockSpec((_HEADS, _HID), lambda i: (0, 0)),
        ],
        out_specs=[
            pl.BlockSpec((_RB, 128), lambda i: (i, 0)),
            pl.BlockSpec((_RB, 128), lambda i: (i, 0)),
        ],
        out_shape=[
            jax.ShapeDtypeStruct((_N, 128), jnp.float32),
            jax.ShapeDtypeStruct((_N, 128), jnp.float32),
        ],
    )(x, W1, att_src1, att_dst1)


def _l2_dense_kernel(y_ref, den_ref, w1_ref, b1_ref, w2_ref, as2_ref, ad2_ref,
                     tabA_ref, tabB_ref, z2p_ref):
    r = y_ref.shape[0]
    rd = (1.0 / (den_ref[0] + den_ref[1] + 1e-16))[:, :_HEADS]
    y3 = y_ref[...].reshape(r, _HEADS, _HID)
    parts = [
        jnp.dot(y3[:, h, :] * rd[:, h:h + 1], w1_ref[:, h * _HID:(h + 1) * _HID],
                preferred_element_type=jnp.float32)
        for h in range(_HEADS)
    ]
    t = jnp.concatenate(parts, axis=1) + b1_ref[...][None, :]
    g = jnp.where(t > 0, t, jnp.exp(jnp.minimum(t, 0.0)) - 1.0)
    z2 = jnp.dot(g, w2_ref[...], preferred_element_type=jnp.float32)
    a2 = jnp.sum(z2 * as2_ref[...], axis=-1, keepdims=True)
    d2 = jnp.sum(z2 * ad2_ref[...], axis=-1, keepdims=True)
    ones = jnp.ones((1, 128), jnp.float32)
    tabA_ref[...] = a2 * ones
    tabB_ref[...] = d2 * ones
    z2p_ref[...] = jnp.concatenate(
        [z2, jnp.zeros((r, 128 - _NCLS), jnp.float32)], axis=1)


def _l2_dense(y, denp1, W1, b1, W2, att_src2, att_dst2):
    return pl.pallas_call(
        _l2_dense_kernel,
        grid=(_NPAD // _RBT,),
        in_specs=[
            pl.BlockSpec((_RBT, _HEADS * _HID), lambda i: (i, 0)),
            pl.BlockSpec((2, _RBT, 16), lambda i: (0, i, 0)),
            pl.BlockSpec((_F, _HEADS * _HID), lambda i: (0, 0)),
            pl.BlockSpec((_HEADS * _HID,), lambda i: (0,)),
            pl.BlockSpec((_HEADS * _HID, _NCLS), lambda i: (0, 0)),
            pl.BlockSpec((1, _NCLS), lambda i: (0, 0)),
            pl.BlockSpec((1, _NCLS), lambda i: (0, 0)),
        ],
        out_specs=[
            pl.BlockSpec((_RBT, 128), lambda i: (i, 0)),
            pl.BlockSpec((_RBT, 128), lambda i: (i, 0)),
            pl.BlockSpec((_RBT, 128), lambda i: (i, 0)),
        ],
        out_shape=[
            jax.ShapeDtypeStruct((_NPAD, 128), jnp.float32),
            jax.ShapeDtypeStruct((_NPAD, 128), jnp.float32),
            jax.ShapeDtypeStruct((_NPAD, 128), jnp.float32),
        ],
    )(y, denp1, W1, b1, W2, att_src2, att_dst2)


def _final_kernel(o_ref, den_ref, b2_ref, out_ref):
    rd = 1.0 / ((den_ref[0] + den_ref[1])[:, 0:1] + 1e-16)
    z = (o_ref[0] + o_ref[1])[:, :_NCLS] * rd + b2_ref[...][None, :]
    m = jnp.max(z, axis=-1, keepdims=True)
    s = jnp.log(jnp.sum(jnp.exp(z - m), axis=-1, keepdims=True))
    out_ref[...] = z - m - s


def _final(out2p, denp2, b2):
    return pl.pallas_call(
        _final_kernel,
        grid=(_NPAD // _RBT,),
        in_specs=[
            pl.BlockSpec((2, _RBT, 16), lambda i: (0, i, 0)),
            pl.BlockSpec((2, _RBT, 16), lambda i: (0, i, 0)),
            pl.BlockSpec((_NCLS,), lambda i: (0,)),
        ],
        out_specs=pl.BlockSpec((_RBT, _NCLS), lambda i: (i, 0)),
        out_shape=jax.ShapeDtypeStruct((_NPAD, _NCLS), jnp.float32),
    )(out2p, denp2, b2)


def _edge_exp_body(tabA, tabB, srcp, dstp, zrows, denp_out,
                   den_sh, srcb, dstb, ab, bb, exb, sem):
    c = lax.axis_index("c")
    s = lax.axis_index("s")
    pltpu.sync_copy(zrows, den_sh.at[pl.ds(s * _RBP, _RBP)])
    plsc.subcore_barrier()

    w = s * 2 + c
    base = w * (_EPAD // 32)

    def block(b, _):
        off = base + b * _EB
        pltpu.sync_copy(srcp.at[pl.ds(off, _EB)], srcb)
        pltpu.sync_copy(dstp.at[pl.ds(off, _EB)], dstb)
        pltpu.async_copy(tabA.at[srcb], ab, sem).wait()
        pltpu.async_copy(tabB.at[dstb], bb, sem).wait()

        def edge(g, _):
            e = ab[g, pl.ds(0, 16)] + bb[g, pl.ds(0, 16)]
            e = jnp.where(e >= 0, e, 0.2 * e)
            exb[g, :] = jnp.exp(e)
            return 0

        lax.fori_loop(0, _EB, edge, 0)
        pltpu.sync_copy(exb, den_sh.at[dstb], add=True)
        return 0

    lax.fori_loop(0, _EPAD // 32 // _EB, block, 0)
    plsc.subcore_barrier()
    pltpu.sync_copy(den_sh.at[pl.ds(s * _RBP, _RBP)],
                    denp_out.at[c, pl.ds(s * _RBP, _RBP)])


def _edge_exp(tabA, tabB, srcp, dstp, zrows):
    return pl.kernel(
        _edge_exp_body,
        out_type=jax.ShapeDtypeStruct((2, _NPAD, 16), jnp.float32),
        mesh=_mesh(),
        scratch_types=[
            pltpu.VMEM_SHARED((_NPAD, 16), jnp.float32),
            pltpu.VMEM((_EB,), jnp.int32),
            pltpu.VMEM((_EB,), jnp.int32),
            pltpu.VMEM((_EB, 128), jnp.float32),
            pltpu.VMEM((_EB, 128), jnp.float32),
            pltpu.VMEM((_EB, 16), jnp.float32),
            pltpu.SemaphoreType.DMA,
        ],
    )(tabA, tabB, srcp, dstp, zrows)


def _msg1_body(xp, srcp, dstp, tabA, tabB, zacc, y_out,
               acc_sh, srcb, dstb, xg, ag, bg, idxb, valb, sem):
    c = lax.axis_index("c")
    s = lax.axis_index("s")

    for p in range(_NCHUNK // 2):
        chunk = c * (_NCHUNK // 2) + p
        lo = chunk * _CROWS
        hi = lo + _CROWS
        rows = _ACC_ROWS // 16
        pltpu.sync_copy(zacc, acc_sh.at[pl.ds(s * rows, rows)])
        plsc.subcore_barrier()

        base = s * (_EPAD // 16)

        def pblock(i, _):
            off = base + i * _G
            pltpu.sync_copy(srcp.at[pl.ds(off, _G)], srcb)
            pltpu.sync_copy(dstp.at[pl.ds(off, _G)], dstb)
            c1 = pltpu.async_copy(xp.at[srcb], xg, sem)
            c2 = pltpu.async_copy(tabA.at[srcb], ag, sem)
            c3 = pltpu.async_copy(tabB.at[dstb], bg, sem)
            c1.wait()
            c2.wait()
            c3.wait()

            for t in range(_G // 16):
                d16 = dstb[pl.ds(t * 16, 16)]
                inr = (d16 >= lo) & (d16 < hi)
                dl8 = jnp.clip(d16 - lo, 0, _CROWS - 1) * 8
                for h in range(_HEADS):
                    idxb[pl.ds(h * _G + t * 16, 16)] = jnp.where(
                        inr, dl8 + h, _ACC_ROWS)

            def edge(g, _):
                e = ag[g, pl.ds(0, 16)] + bg[g, pl.ds(0, 16)]
                e = jnp.where(e >= 0, e, 0.2 * e)
                ex = jnp.exp(e)
                for h in range(_HEADS):
                    sc = ex[h]
                    for k in range(_F // 16):
                        valb[h * _G + g, pl.ds(k * 16, 16)] = (
                            sc * xg[g, pl.ds(k * 16, 16)])
                return 0

            lax.fori_loop(0, _G, edge, 0)
            pltpu.sync_copy(valb, acc_sh.at[idxb], add=True)
            return 0

        lax.fori_loop(0, _EPAD // 16 // _G, pblock, 0)
        plsc.subcore_barrier()
        pltpu.sync_copy(acc_sh.at[pl.ds(s * rows, rows)],
                        y_out.at[pl.ds(chunk * _ACC_ROWS + s * rows, rows)])
        plsc.subcore_barrier()


def _msg1(xp, srcp, dstp, tabA, tabB, zacc):
    return pl.kernel(
        _msg1_body,
        out_type=jax.ShapeDtypeStruct((_YROWS, _F), jnp.float32),
        mesh=_mesh(),
        scratch_types=[
            pltpu.VMEM_SHARED((_ACC_ROWS + 16, _F), jnp.float32),
            pltpu.VMEM((_G,), jnp.int32),
            pltpu.VMEM((_G,), jnp.int32),
            pltpu.VMEM((_G, _F), jnp.float32),
            pltpu.VMEM((_G, 128), jnp.float32),
            pltpu.VMEM((_G, 128), jnp.float32),
            pltpu.VMEM((_HEADS * _G,), jnp.int32),
            pltpu.VMEM((_HEADS * _G, _F), jnp.float32),
            pltpu.SemaphoreType.DMA,
        ],
    )(xp, srcp, dstp, tabA, tabB, zacc)


def _msg2_body(z2p, srcp, dstp, tabA, tabB, zrows, out2p,
               o_sh, srcb, dstb, zg, ag, bg, valb, sem):
    c = lax.axis_index("c")
    s = lax.axis_index("s")
    pltpu.sync_copy(zrows, o_sh.at[pl.ds(s * _RBP, _RBP)])
    plsc.subcore_barrier()

    w = s * 2 + c
    base = w * (_EPAD // 32)

    def block(b, _):
        off = base + b * _EB2
        pltpu.sync_copy(srcp.at[pl.ds(off, _EB2)], srcb)
        pltpu.sync_copy(dstp.at[pl.ds(off, _EB2)], dstb)
        pltpu.async_copy(z2p.at[srcb], zg, sem).wait()
        pltpu.async_copy(tabA.at[srcb], ag, sem).wait()
        pltpu.async_copy(tabB.at[dstb], bg, sem).wait()

        def edge(g, _):
            e = ag[g, pl.ds(0, 16)] + bg[g, pl.ds(0, 16)]
            e = jnp.where(e >= 0, e, 0.2 * e)
            valb[g, :] = jnp.exp(e) * zg[g, pl.ds(0, 16)]
            return 0

        lax.fori_loop(0, _EB2, edge, 0)
        pltpu.sync_copy(valb, o_sh.at[dstb], add=True)
        return 0

    lax.fori_loop(0, _EPAD // 32 // _EB2, block, 0)
    plsc.subcore_barrier()
    pltpu.sync_copy(o_sh.at[pl.ds(s * _RBP, _RBP)],
                    out2p.at[c, pl.ds(s * _RBP, _RBP)])


def _msg2(z2p, srcp, dstp, tabA, tabB, zrows):
    return pl.kernel(
        _msg2_body,
        out_type=jax.ShapeDtypeStruct((2, _NPAD, 16), jnp.float32),
        mesh=_mesh(),
        scratch_types=[
            pltpu.VMEM_SHARED((_NPAD, 16), jnp.float32),
            pltpu.VMEM((_EB2,), jnp.int32),
            pltpu.VMEM((_EB2,), jnp.int32),
            pltpu.VMEM((_EB2, 128), jnp.float32),
            pltpu.VMEM((_EB2, 128), jnp.float32),
            pltpu.VMEM((_EB2, 128), jnp.float32),
            pltpu.VMEM((_EB2, 16), jnp.float32),
            pltpu.SemaphoreType.DMA,
        ],
    )(z2p, srcp, dstp, tabA, tabB, zrows)


def kernel(x, edge_index, W1, att_src1, att_dst1, b1, W2, att_src2, att_dst2, b2):
    f32 = jnp.float32
    i32 = jnp.int32
    loop = jnp.arange(_N, dtype=i32)
    srcp = jnp.concatenate([edge_index[0].astype(i32), loop,
                            jnp.zeros((_EPAD - _E1,), i32)])
    dstp = jnp.concatenate([edge_index[1].astype(i32), loop,
                            jnp.full((_EPAD - _E1,), _N, i32)])

    xp = jnp.pad(x, ((0, _NPAD - _N), (0, 0)))
    zrows = jnp.zeros((_RBP, 16), f32)
    zacc = jnp.zeros((_ACC_ROWS // 16, _F), f32)

    as1, ad1 = _l1_alpha(x, W1, att_src1, att_dst1)
    tabA1 = jnp.pad(as1, ((0, _NPAD - _N), (0, 0)))
    tabB1 = jnp.pad(ad1, ((0, _NPAD - _N), (0, 0)))
    denp1 = _edge_exp(tabA1, tabB1, srcp, dstp, zrows)
    y = _msg1(xp, srcp, dstp, tabA1, tabB1, zacc)
    y2 = y.reshape(_NCHUNK * _CROWS, _HEADS * _HID)[:_NPAD]

    tabA2, tabB2, z2p = _l2_dense(y2, denp1, W1, b1, W2, att_src2, att_dst2)
    denp2 = _edge_exp(tabA2, tabB2, srcp, dstp, zrows)
    out2p = _msg2(z2p, srcp, dstp, tabA2, tabB2, zrows)

    return _final(out2p, denp2, b2)[:_N]
